# Initial kernel scaffold; baseline (speedup 1.0000x reference)
import jax
import jax.numpy as jnp
from jax.experimental import pallas as pl


def kernel(relation_matrix, embeddings_table):
    raise NotImplementedError("write your pallas kernel here")


# speedup vs baseline: 1.5810x; 1.5810x over previous
import functools

import jax
import jax.numpy as jnp
from jax import lax
from jax.experimental import pallas as pl
from jax.experimental.pallas import tpu as pltpu
from jax.experimental.pallas import tpu_sc as plsc

NUM_UNITS = 768
NUM_REL = 129


def _zero_row0_body(table_ref, out_ref):
    rows = lax.broadcasted_iota(jnp.int32, table_ref.shape, 0)
    out_ref[...] = jnp.where(rows == 0, jnp.float32(0.0), table_ref[...])


def _zero_row0(table):
    return pl.pallas_call(
        _zero_row0_body,
        out_shape=jax.ShapeDtypeStruct(table.shape, table.dtype),
    )(table)


@functools.lru_cache(maxsize=None)
def _make_sc_gather(B, D):
    info = plsc.get_sparse_core_info()
    NC, NS = info.num_cores, info.num_subcores
    NW = NC * NS
    b_per_w = B // NW
    C = 64
    n_chunks = b_per_w // C
    assert b_per_w % C == 0 and n_chunks % 2 == 0

    mesh = plsc.VectorSubcoreMesh(core_axis_name="c", subcore_axis_name="s")

    @functools.partial(
        pl.kernel,
        mesh=mesh,
        out_type=jax.ShapeDtypeStruct((B, D), jnp.float32),
        scratch_types=[
            pltpu.VMEM((b_per_w,), jnp.int32),
            pltpu.VMEM((C, D), jnp.float32),
            pltpu.VMEM((C, D), jnp.float32),
            pltpu.SemaphoreType.DMA,
            pltpu.SemaphoreType.DMA,
            pltpu.SemaphoreType.DMA,
            pltpu.SemaphoreType.DMA,
        ],
    )
    def gather_kernel(table_hbm, idx_hbm, out_hbm, idx_v, rows0, rows1,
                      gsem0, gsem1, wsem0, wsem1):
        wid = lax.axis_index("s") * NC + lax.axis_index("c")
        base = wid * b_per_w
        pltpu.sync_copy(idx_hbm.at[pl.ds(base, b_per_w)], idx_v)

        rows = (rows0, rows1)
        gsem = (gsem0, gsem1)
        wsem = (wsem0, wsem1)

        def g_copy(c, b):
            return pltpu.make_async_copy(
                table_hbm.at[idx_v.at[pl.ds(c * C, C)]], rows[b], gsem[b])

        def w_copy(c, b):
            return pltpu.make_async_copy(
                rows[b], out_hbm.at[pl.ds(base + c * C, C)], wsem[b])

        g_copy(0, 0).start()

        def loop_body(g):
            for b in range(2):
                c = g + b
                nb = 1 - b
                g_copy(c, b).wait()
                w_copy(c, b).start()

                @pl.when(c >= 1)
                def _():
                    w_copy(c - 1, nb).wait()

                @pl.when(c + 1 < n_chunks)
                def _():
                    g_copy(c + 1, nb).start()

        pl.loop(0, n_chunks, step=2)(loop_body)
        w_copy(n_chunks - 1, (n_chunks - 1) % 2).wait()

    return gather_kernel


def kernel(relation_matrix, embeddings_table):
    bsz, seq, seq2 = relation_matrix.shape
    num_units = embeddings_table.shape[1]
    idx = relation_matrix.reshape(-1)
    table = _zero_row0(embeddings_table)
    out = _make_sc_gather(idx.shape[0], num_units)(table, idx)
    return out.reshape(bsz, seq, seq2, num_units)
